# initial kernel scaffold (unmeasured)
import jax
import jax.numpy as jnp
from jax import lax
from jax.experimental import pallas as pl
from jax.experimental.pallas import tpu as pltpu

T = 512
D = 256
CH = 32
N_FULL = T // CH
NSEM = N_FULL + 1


def _chunked(count, full_fn, tail_fn):
    for c in range(N_FULL):
        @pl.when((c + 1) * CH <= count)
        def _(c=c):
            full_fn(c)

    @pl.when(count % CH != 0)
    def _():
        tail_fn()


def kernel(x, dest):
    my_y = lax.axis_index("y")
    is0 = my_y == 0

    order = jnp.argsort(dest, stable=True)
    sorted_x = jnp.take(x, order, axis=0)
    l0 = jnp.sum((dest == 0).astype(jnp.int32))

    n_send = jnp.where(is0, T - l0, l0)
    send_off = jnp.where(is0, l0, 0)
    n_keep = T - n_send
    keep_off = jnp.where(is0, 0, l0)
    n_recv = T - n_keep
    out_keep_off = jnp.where(is0, 0, n_recv)
    recv_off = jnp.where(is0, n_keep, 0)
    dst_off = jnp.where(is0, 0, T - n_send)

    params = jnp.stack(
        [n_send, send_off, dst_off, n_keep, keep_off, out_keep_off,
         n_recv, recv_off]
    ).astype(jnp.int32)

    def body(x_ref, p_ref, out_ref, send_sems, recv_sems, copy_sems):
        mx = lax.axis_index("x")
        my = lax.axis_index("y")
        mz = lax.axis_index("z")
        peer = (mx, 1 - my, mz)

        ns, s_off, d_off = p_ref[0], p_ref[1], p_ref[2]
        nk, k_off, ok_off = p_ref[3], p_ref[4], p_ref[5]
        nr, r_off = p_ref[6], p_ref[7]

        bar = pltpu.get_barrier_semaphore()
        pl.semaphore_signal(bar, inc=1, device_id=peer,
                            device_id_type=pl.DeviceIdType.MESH)
        pl.semaphore_wait(bar, 1)

        def rdma(src_start, dst_start, sem_idx):
            return pltpu.make_async_remote_copy(
                src_ref=x_ref.at[pl.ds(src_start, CH)],
                dst_ref=out_ref.at[pl.ds(dst_start, CH)],
                send_sem=send_sems.at[sem_idx],
                recv_sem=recv_sems.at[sem_idx],
                device_id=peer,
                device_id_type=pl.DeviceIdType.MESH,
            )

        def lcopy(src_start, dst_start, sem_idx):
            return pltpu.make_async_copy(
                x_ref.at[pl.ds(src_start, CH)],
                out_ref.at[pl.ds(dst_start, CH)],
                copy_sems.at[sem_idx],
            )

        _chunked(ns,
                 lambda c: rdma(s_off + c * CH, d_off + c * CH, c).start(),
                 lambda: rdma(s_off + ns - CH, d_off + ns - CH,
                              N_FULL).start())
        _chunked(nk,
                 lambda c: lcopy(k_off + c * CH, ok_off + c * CH, c).start(),
                 lambda: lcopy(k_off + nk - CH, ok_off + nk - CH,
                               N_FULL).start())

        _chunked(nr,
                 lambda c: rdma(0, r_off + c * CH, c).wait_recv(),
                 lambda: rdma(0, r_off + nr - CH, N_FULL).wait_recv())
        _chunked(nk,
                 lambda c: lcopy(k_off + c * CH, ok_off + c * CH, c).wait(),
                 lambda: lcopy(k_off + nk - CH, ok_off + nk - CH,
                               N_FULL).wait())
        _chunked(ns,
                 lambda c: rdma(s_off, d_off, c).wait_send(),
                 lambda: rdma(s_off, d_off, N_FULL).wait_send())

    return pl.pallas_call(
        body,
        out_shape=jax.ShapeDtypeStruct((T, D), jnp.float32),
        in_specs=[
            pl.BlockSpec(memory_space=pltpu.VMEM),
            pl.BlockSpec(memory_space=pltpu.SMEM),
        ],
        out_specs=pl.BlockSpec(memory_space=pltpu.VMEM),
        scratch_shapes=[
            pltpu.SemaphoreType.DMA((NSEM,)),
            pltpu.SemaphoreType.DMA((NSEM,)),
            pltpu.SemaphoreType.DMA((NSEM,)),
        ],
        compiler_params=pltpu.CompilerParams(collective_id=0),
    )(sorted_x, params)


# baseline (device time: 10851 ns/iter reference)
import jax
import jax.numpy as jnp
from jax import lax
from jax.experimental import pallas as pl
from jax.experimental.pallas import tpu as pltpu

T = 512
D = 256
CH = 32
N_CH = T // CH


def _body(x_ref, dest_ref, out_ref, send_buf, recv_buf, send_sems,
          recv_sems):
    mx = lax.axis_index("x")
    my = lax.axis_index("y")
    mz = lax.axis_index("z")
    peer = (mx, 1 - my, mz)

    bar = pltpu.get_barrier_semaphore()
    pl.semaphore_signal(bar, inc=1, device_id=peer,
                        device_id_type=pl.DeviceIdType.MESH)
    pl.semaphore_wait(bar, 1)

    x = x_ref[...]
    d = dest_ref[...].astype(jnp.float32)
    myf = my.astype(jnp.float32)
    send_m = jnp.where(d != myf, 1.0, 0.0)
    keep_m = 1.0 - send_m
    ns_f = jnp.sum(send_m)
    ns_i = jnp.sum(send_m.astype(jnp.int32))

    p = lax.broadcasted_iota(jnp.int32, (T, T), 0).astype(jnp.float32)
    icol = lax.broadcasted_iota(jnp.int32, (T, T), 1).astype(jnp.float32)

    tri = jnp.where(icol > p, 1.0, 0.0)
    send_rank = jnp.dot(send_m, tri, preferred_element_type=jnp.float32)
    C = jnp.where((send_rank == p) & (send_m > 0.5), 1.0, 0.0)
    send_buf[...] = jnp.dot(C, x, preferred_element_type=jnp.float32)

    def rdma(c):
        return pltpu.make_async_remote_copy(
            src_ref=send_buf.at[pl.ds(c * CH, CH)],
            dst_ref=recv_buf.at[pl.ds(c * CH, CH)],
            send_sem=send_sems.at[c],
            recv_sem=recv_sems.at[c],
            device_id=peer,
            device_id_type=pl.DeviceIdType.MESH,
        )

    for c in range(N_CH):
        @pl.when(c * CH < ns_i)
        def _(c=c):
            rdma(c).start()

    keep_rank = jnp.dot(keep_m, tri, preferred_element_type=jnp.float32)
    kept_first = jnp.where(my == 0, 0.0, ns_f)
    A = jnp.where((kept_first + keep_rank == p) & (keep_m > 0.5), 1.0, 0.0)
    recv_start = jnp.where(my == 0, T - ns_f, 0.0)
    B = jnp.where((recv_start + icol == p) & (icol < ns_f), 1.0, 0.0)
    ax = jnp.dot(A, x, preferred_element_type=jnp.float32)

    for c in range(N_CH):
        @pl.when(c * CH < ns_i)
        def _(c=c):
            rdma(c).wait_recv()

    out_ref[...] = ax + jnp.dot(B, recv_buf[...],
                                preferred_element_type=jnp.float32)

    for c in range(N_CH):
        @pl.when(c * CH < ns_i)
        def _(c=c):
            rdma(c).wait_send()


def kernel(x, dest):
    return pl.pallas_call(
        _body,
        out_shape=jax.ShapeDtypeStruct((T, D), jnp.float32),
        in_specs=[
            pl.BlockSpec(memory_space=pltpu.VMEM),
            pl.BlockSpec(memory_space=pltpu.VMEM),
        ],
        out_specs=pl.BlockSpec(memory_space=pltpu.VMEM),
        scratch_shapes=[
            pltpu.VMEM((T, D), jnp.float32),
            pltpu.VMEM((T, D), jnp.float32),
            pltpu.SemaphoreType.DMA((N_CH,)),
            pltpu.SemaphoreType.DMA((N_CH,)),
        ],
        compiler_params=pltpu.CompilerParams(collective_id=0),
    )(x, dest.reshape(1, T))


# device time: 9450 ns/iter; 1.1483x vs baseline; 1.1483x over previous
import jax
import jax.numpy as jnp
from jax import lax
from jax.experimental import pallas as pl
from jax.experimental.pallas import tpu as pltpu

T = 512
D = 256
CH = 32
N_CH = T // CH


def _body(x_ref, dest_ref, out_ref, send_buf, recv_buf, send_sems,
          recv_sems):
    mx = lax.axis_index("x")
    my = lax.axis_index("y")
    mz = lax.axis_index("z")
    peer = (mx, 1 - my, mz)

    bar = pltpu.get_barrier_semaphore()
    pl.semaphore_signal(bar, inc=1, device_id=peer,
                        device_id_type=pl.DeviceIdType.MESH)
    pl.semaphore_wait(bar, 1)

    x = x_ref[...].astype(jnp.bfloat16)
    d = dest_ref[...].astype(jnp.float32)
    myf = my.astype(jnp.float32)
    send_m = jnp.where(d != myf, 1.0, 0.0)
    keep_m = 1.0 - send_m
    ns_f = jnp.sum(send_m)
    ns_i = jnp.sum(send_m.astype(jnp.int32))

    p = lax.broadcasted_iota(jnp.int32, (T, T), 0).astype(jnp.float32)
    icol = lax.broadcasted_iota(jnp.int32, (T, T), 1).astype(jnp.float32)

    tri = jnp.where(icol > p, 1.0, 0.0)
    send_rank = jnp.dot(send_m, tri, preferred_element_type=jnp.float32)
    C = jnp.where((send_rank == p) & (send_m > 0.5), 1.0, 0.0).astype(
        jnp.bfloat16)
    send_buf[...] = jnp.dot(
        C, x, preferred_element_type=jnp.float32).astype(jnp.bfloat16)

    def rdma(c):
        return pltpu.make_async_remote_copy(
            src_ref=send_buf.at[pl.ds(c * CH, CH)],
            dst_ref=recv_buf.at[pl.ds(c * CH, CH)],
            send_sem=send_sems.at[c],
            recv_sem=recv_sems.at[c],
            device_id=peer,
            device_id_type=pl.DeviceIdType.MESH,
        )

    for c in range(N_CH):
        @pl.when(c * CH < ns_i)
        def _(c=c):
            rdma(c).start()

    keep_rank = jnp.dot(keep_m, tri, preferred_element_type=jnp.float32)
    kept_first = jnp.where(my == 0, 0.0, ns_f)
    A = jnp.where((kept_first + keep_rank == p) & (keep_m > 0.5),
                  1.0, 0.0).astype(jnp.bfloat16)
    recv_start = jnp.where(my == 0, T - ns_f, 0.0)
    B = jnp.where((recv_start + icol == p) & (icol < ns_f),
                  1.0, 0.0).astype(jnp.bfloat16)
    ax = jnp.dot(A, x, preferred_element_type=jnp.float32)

    for c in range(N_CH):
        @pl.when(c * CH < ns_i)
        def _(c=c):
            rdma(c).wait_recv()

    out_ref[...] = ax + jnp.dot(B, recv_buf[...],
                                preferred_element_type=jnp.float32)

    for c in range(N_CH):
        @pl.when(c * CH < ns_i)
        def _(c=c):
            rdma(c).wait_send()


def kernel(x, dest):
    return pl.pallas_call(
        _body,
        out_shape=jax.ShapeDtypeStruct((T, D), jnp.float32),
        in_specs=[
            pl.BlockSpec(memory_space=pltpu.VMEM),
            pl.BlockSpec(memory_space=pltpu.VMEM),
        ],
        out_specs=pl.BlockSpec(memory_space=pltpu.VMEM),
        scratch_shapes=[
            pltpu.VMEM((T, D), jnp.bfloat16),
            pltpu.VMEM((T, D), jnp.bfloat16),
            pltpu.SemaphoreType.DMA((N_CH,)),
            pltpu.SemaphoreType.DMA((N_CH,)),
        ],
        compiler_params=pltpu.CompilerParams(collective_id=0),
    )(x, dest.reshape(1, T))


# device time: 9411 ns/iter; 1.1530x vs baseline; 1.0041x over previous
import jax
import jax.numpy as jnp
from jax import lax
from jax.experimental import pallas as pl
from jax.experimental.pallas import tpu as pltpu

T = 512
D = 256
CH = 32
N_CH = T // CH


def _body(x_ref, dest_ref, out_ref, send_buf, recv_buf, send_sems,
          recv_sems):
    mx = lax.axis_index("x")
    my = lax.axis_index("y")
    mz = lax.axis_index("z")
    peer = (mx, 1 - my, mz)

    recv_buf[...] = jnp.zeros((T, D), jnp.bfloat16)

    bar = pltpu.get_barrier_semaphore()
    pl.semaphore_signal(bar, inc=1, device_id=peer,
                        device_id_type=pl.DeviceIdType.MESH)
    pl.semaphore_wait(bar, 1)

    x = x_ref[...].astype(jnp.bfloat16)
    d = dest_ref[...].astype(jnp.float32)
    myf = my.astype(jnp.float32)
    send_m = jnp.where(d != myf, 1.0, 0.0)
    keep_m = 1.0 - send_m
    ns_f = jnp.sum(send_m)
    ns_i = jnp.sum(send_m.astype(jnp.int32))

    p = lax.broadcasted_iota(jnp.int32, (T, T), 0).astype(jnp.float32)
    icol = lax.broadcasted_iota(jnp.int32, (T, T), 1).astype(jnp.float32)

    tri = jnp.where(icol > p, 1.0, 0.0)
    send_rank = jnp.dot(send_m, tri, preferred_element_type=jnp.float32)
    C = jnp.where((send_rank == p) & (send_m > 0.5), 1.0, 0.0).astype(
        jnp.bfloat16)
    send_buf[...] = jnp.dot(
        C, x, preferred_element_type=jnp.float32).astype(jnp.bfloat16)

    def rdma(c):
        return pltpu.make_async_remote_copy(
            src_ref=send_buf.at[pl.ds(c * CH, CH)],
            dst_ref=recv_buf.at[pl.ds(c * CH, CH)],
            send_sem=send_sems.at[c],
            recv_sem=recv_sems.at[c],
            device_id=peer,
            device_id_type=pl.DeviceIdType.MESH,
        )

    for c in range(N_CH):
        @pl.when(c * CH < ns_i)
        def _(c=c):
            rdma(c).start()

    keep_rank = jnp.dot(keep_m, tri, preferred_element_type=jnp.float32)
    kept_first = jnp.where(my == 0, 0.0, ns_f)
    A = jnp.where((kept_first + keep_rank == p) & (keep_m > 0.5),
                  1.0, 0.0).astype(jnp.bfloat16)
    recv_start = jnp.where(my == 0, T - ns_f, 0.0)
    B = jnp.where((recv_start + icol == p) & (icol < ns_f),
                  1.0, 0.0).astype(jnp.bfloat16)
    ax = jnp.dot(A, x, preferred_element_type=jnp.float32)

    for c in range(N_CH):
        @pl.when(c * CH < ns_i)
        def _(c=c):
            rdma(c).wait_recv()

    out_ref[...] = ax + jnp.dot(B, recv_buf[...],
                                preferred_element_type=jnp.float32)

    for c in range(N_CH):
        @pl.when(c * CH < ns_i)
        def _(c=c):
            rdma(c).wait_send()


def kernel(x, dest):
    return pl.pallas_call(
        _body,
        out_shape=jax.ShapeDtypeStruct((T, D), jnp.float32),
        in_specs=[
            pl.BlockSpec(memory_space=pltpu.VMEM),
            pl.BlockSpec(memory_space=pltpu.VMEM),
        ],
        out_specs=pl.BlockSpec(memory_space=pltpu.VMEM),
        scratch_shapes=[
            pltpu.VMEM((T, D), jnp.bfloat16),
            pltpu.VMEM((T, D), jnp.bfloat16),
            pltpu.SemaphoreType.DMA((N_CH,)),
            pltpu.SemaphoreType.DMA((N_CH,)),
        ],
        compiler_params=pltpu.CompilerParams(collective_id=0),
    )(x, dest.reshape(1, T))
